# baseline (speedup 1.0000x reference)
import jax
import jax.numpy as jnp
from jax import lax
from jax.experimental import pallas as pl
from jax.experimental.pallas import tpu as pltpu
from jax.experimental.pallas import tpu_sc as plsc

N_NODES = 10000
N_EDGES = 320000
D = 128
NPAD = 10240
NDIS = 10016
DEAD = 10008
NC = 2
NS = 16
NW = NC * NS
CHUNK = 128
NCH = 80
EPT = NCH * CHUNK
E_PAD = NW * EPT
NSLOT = 2
ISLOT = 4
DSLOT = 4

_MESH = plsc.VectorSubcoreMesh(core_axis_name="c", subcore_axis_name="s")
_SC_PARAMS = pltpu.CompilerParams(needs_layout_passes=False)


def _deg_body(dst_hbm, zs_hbm, deg_hbm, deg_sp, didx, ones_v,
              s0, s1, s2, s3):
  cid = lax.axis_index("c")
  sid = lax.axis_index("s")
  wid = cid * NS + sid
  sems = (s0, s1, s2, s3)

  pltpu.sync_copy(zs_hbm.at[pl.ds(sid * 640, 640)],
                  deg_sp.at[pl.ds(sid * 640, 640)])
  pltpu.sync_copy(dst_hbm.at[wid], didx)
  for i in range(8):
    ones_v[pl.ds(i * 16, 16)] = jnp.full((16,), 1.0, jnp.float32)
  plsc.subcore_barrier()

  def body(j, carry):
    descs = []
    for p in range(DSLOT):
      descs.append(pltpu.async_copy(
          ones_v, deg_sp.at[didx.at[j * DSLOT + p]], sems[p], add=True))
    for d in descs:
      d.wait()
    return carry

  lax.fori_loop(0, NCH // DSLOT, body, 0)

  plsc.subcore_barrier()
  pltpu.sync_copy(deg_sp.at[pl.ds(sid * 640, 640)],
                  deg_hbm.at[cid, pl.ds(sid * 640, 640)])


_deg = pl.kernel(
    _deg_body,
    out_type=jax.ShapeDtypeStruct((NC, NPAD), jnp.float32),
    mesh=_MESH,
    compiler_params=_SC_PARAMS,
    scratch_types=[
        pltpu.VMEM_SHARED((NPAD,), jnp.float32),
        pltpu.VMEM((NCH, CHUNK), jnp.int32),
        pltpu.VMEM((CHUNK,), jnp.float32),
        pltpu.SemaphoreType.DMA,
        pltpu.SemaphoreType.DMA,
        pltpu.SemaphoreType.DMA,
        pltpu.SemaphoreType.DMA,
    ],
)


def _proj_body(x_ref, w_ref, degp_ref, g_ref, dis_ref):
  deg = degp_ref[0] + degp_ref[1] + 1.0
  dis = lax.rsqrt(deg)
  dis_ref[...] = dis
  h = jnp.dot(x_ref[...], w_ref[...], preferred_element_type=jnp.float32)
  g_ref[...] = dis * h


def _proj(x, W1, degp):
  bm = 2000
  return pl.pallas_call(
      _proj_body,
      grid=(N_NODES // bm,),
      in_specs=[
          pl.BlockSpec((bm, D), lambda i: (i, 0)),
          pl.BlockSpec((D, D), lambda i: (0, 0)),
          pl.BlockSpec((NC, bm, 1), lambda i: (0, i, 0)),
      ],
      out_specs=[
          pl.BlockSpec((bm, D), lambda i: (i, 0)),
          pl.BlockSpec((bm, 1), lambda i: (i, 0)),
      ],
      out_shape=[
          jax.ShapeDtypeStruct((N_NODES, D), jnp.float32),
          jax.ShapeDtypeStruct((N_NODES, 1), jnp.float32),
      ],
  )(x, W1, degp)


def _edge_body(src_hbm, dst_hbm, dis_hbm, g_hbm, zrow_hbm, zs_hbm,
               sp_hbm, accp_hbm,
               acc_sp, s_sp, dis_v, srcb, dstb, val_v, row_v,
               g0, g1, t0, t1, i0, i1, i2, i3):
  cid = lax.axis_index("c")
  sid = lax.axis_index("s")
  wid = cid * NS + sid
  gsem = (g0, g1)
  ssem = (t0, t1)
  isem = (i0, i1, i2, i3)

  def issue_idx(c, q):
    pltpu.async_copy(src_hbm.at[wid, c], srcb.at[q], isem[q])
    pltpu.async_copy(dst_hbm.at[wid, c], dstb.at[q], isem[q])

  def wait_idx(c, q):
    pltpu.make_async_copy(src_hbm.at[wid, c], srcb.at[q], isem[q]).wait()
    pltpu.make_async_copy(dst_hbm.at[wid, c], dstb.at[q], isem[q]).wait()

  pltpu.sync_copy(zrow_hbm.at[pl.ds(sid * 640, 640)],
                  acc_sp.at[pl.ds(sid * 640, 640)])
  pltpu.sync_copy(zs_hbm.at[pl.ds(sid * 640, 640)],
                  s_sp.at[pl.ds(sid * 640, 640)])
  pltpu.sync_copy(dis_hbm, dis_v)
  plsc.subcore_barrier()

  for q in range(ISLOT):
    issue_idx(q, q)
  for p in range(NSLOT):
    wait_idx(p, p)
    pltpu.async_copy(g_hbm.at[srcb.at[p]], row_v.at[p], gsem[p])

  def body(j, carry):
    for u in range(ISLOT):
      c = j * ISLOT + u
      p = u % NSLOT
      q = u
      pltpu.make_async_copy(
          g_hbm.at[srcb.at[q]], row_v.at[p], gsem[p]).wait()
      sdesc = pltpu.async_copy(
          row_v.at[p], acc_sp.at[dstb.at[q]], ssem[p], add=True)
      dstb_row = dstb.at[q]
      for k in range(8):
        dv = dstb_row[pl.ds(k * 16, 16)]
        val_v[pl.ds(k * 16, 16)] = plsc.load_gather(dis_v, [dv])
      pltpu.sync_copy(val_v, s_sp.at[srcb.at[q]], add=True)
      sdesc.wait()

      @pl.when(c + NSLOT < NCH)
      def _():
        qn = (u + NSLOT) % ISLOT
        wait_idx(c + NSLOT, qn)
        pltpu.async_copy(g_hbm.at[srcb.at[qn]], row_v.at[p], gsem[p])

      @pl.when(c + ISLOT < NCH)
      def _():
        issue_idx(c + ISLOT, q)
    return carry

  lax.fori_loop(0, NCH // ISLOT, body, 0)

  plsc.subcore_barrier()
  pltpu.sync_copy(acc_sp.at[pl.ds(sid * 640, 640)],
                  accp_hbm.at[cid, pl.ds(sid * 640, 640)])
  pltpu.sync_copy(s_sp.at[pl.ds(sid * 640, 640)],
                  sp_hbm.at[cid, pl.ds(sid * 640, 640)])


_edge_pass = pl.kernel(
    _edge_body,
    out_type=(
        jax.ShapeDtypeStruct((NC, NPAD), jnp.float32),
        jax.ShapeDtypeStruct((NC, NPAD, D), jnp.float32),
    ),
    mesh=_MESH,
    compiler_params=_SC_PARAMS,
    scratch_types=[
        pltpu.VMEM_SHARED((NPAD, D), jnp.float32),
        pltpu.VMEM_SHARED((NPAD,), jnp.float32),
        pltpu.VMEM((NDIS,), jnp.float32),
        pltpu.VMEM((ISLOT, CHUNK), jnp.int32),
        pltpu.VMEM((ISLOT, CHUNK), jnp.int32),
        pltpu.VMEM((CHUNK,), jnp.float32),
        pltpu.VMEM((NSLOT, CHUNK, D), jnp.float32),
        pltpu.SemaphoreType.DMA,
        pltpu.SemaphoreType.DMA,
        pltpu.SemaphoreType.DMA,
        pltpu.SemaphoreType.DMA,
        pltpu.SemaphoreType.DMA,
        pltpu.SemaphoreType.DMA,
        pltpu.SemaphoreType.DMA,
        pltpu.SemaphoreType.DMA,
    ],
)


def _head_body(dis_ref, g_ref, accp_ref, sp_ref, b1_ref, w2_ref, b2_ref,
               wfc_ref, bfc_ref, out_ref, vacc):
  i = pl.program_id(0)
  dis = dis_ref[...]
  acc = accp_ref[0] + accp_ref[1]
  z1 = dis * (acc + g_ref[...]) + b1_ref[...]
  r1 = jnp.maximum(z1, 0.0)
  s = sp_ref[0] + sp_ref[1]
  w = dis * (dis + s)
  contrib = jnp.sum(w * r1, axis=0, keepdims=True)

  @pl.when(i == 0)
  def _():
    vacc[...] = jnp.zeros_like(vacc)

  vacc[...] += contrib

  @pl.when(i == pl.num_programs(0) - 1)
  def _():
    pooled = jnp.dot(vacc[...] * (1.0 / N_NODES), w2_ref[...],
                     preferred_element_type=jnp.float32) + b2_ref[...]
    out_ref[...] = jnp.dot(pooled, wfc_ref[...],
                           preferred_element_type=jnp.float32) + bfc_ref[...]


def _head(dis_col, g, accp, sp, b1, W2, b2, Wfc, bfc):
  bm = 2000
  return pl.pallas_call(
      _head_body,
      grid=(N_NODES // bm,),
      in_specs=[
          pl.BlockSpec((bm, 1), lambda i: (i, 0)),
          pl.BlockSpec((bm, D), lambda i: (i, 0)),
          pl.BlockSpec((NC, bm, D), lambda i: (0, i, 0)),
          pl.BlockSpec((NC, bm, 1), lambda i: (0, i, 0)),
          pl.BlockSpec((1, D), lambda i: (0, 0)),
          pl.BlockSpec((D, D), lambda i: (0, 0)),
          pl.BlockSpec((1, D), lambda i: (0, 0)),
          pl.BlockSpec((D, D), lambda i: (0, 0)),
          pl.BlockSpec((1, D), lambda i: (0, 0)),
      ],
      out_specs=pl.BlockSpec((1, D), lambda i: (0, 0)),
      out_shape=jax.ShapeDtypeStruct((1, D), jnp.float32),
      scratch_shapes=[pltpu.VMEM((1, D), jnp.float32)],
  )(dis_col, g, accp, sp, b1, W2, b2, Wfc, bfc)


def kernel(x, edge_index, W1, b1, W2, b2, Wfc, bfc):
  src = edge_index[0].astype(jnp.int32)
  dst = edge_index[1].astype(jnp.int32)
  npad_e = E_PAD - N_EDGES
  src_p = jnp.concatenate(
      [src, jnp.zeros((npad_e,), jnp.int32)]).reshape(NW, NCH, CHUNK)
  dst_p = jnp.concatenate(
      [dst, jnp.full((npad_e,), DEAD, jnp.int32)]).reshape(NW, NCH, CHUNK)
  zrow = jnp.zeros((NPAD, D), jnp.float32)
  zs = jnp.zeros((NPAD,), jnp.float32)

  degp = _deg(dst_p, zs)
  g, dis_col = _proj(x, W1, degp.reshape(NC, NPAD, 1))
  dis_pad = jnp.concatenate(
      [dis_col.reshape(N_NODES), jnp.zeros((NDIS - N_NODES,), jnp.float32)])
  sp, accp = _edge_pass(src_p, dst_p, dis_pad, g, zrow, zs)
  sp_col = sp.reshape(NC, NPAD, 1)
  b1r = b1.reshape(1, D)
  b2r = b2.reshape(1, D)
  bfcr = bfc.reshape(1, D)
  return _head(dis_col, g, accp, sp_col, b1r, W2, b2r, Wfc, bfcr)


# speedup vs baseline: 1.0004x; 1.0004x over previous
import jax
import jax.numpy as jnp
from jax import lax
from jax.experimental import pallas as pl
from jax.experimental.pallas import tpu as pltpu
from jax.experimental.pallas import tpu_sc as plsc

N_NODES = 10000
N_EDGES = 320000
D = 128
NPAD = 10240
NDIS = 10016
DEAD = 10008
NC = 2
NS = 16
NW = NC * NS
CHUNK = 128
NCH = 80
EPT = NCH * CHUNK
E_PAD = NW * EPT
NSLOT = 2
ISLOT = 4
DSLOT = 4

_MESH = plsc.VectorSubcoreMesh(core_axis_name="c", subcore_axis_name="s")
_SC_PARAMS = pltpu.CompilerParams(needs_layout_passes=False)


def _deg_body(dst_hbm, zs_hbm, deg_hbm, deg_sp, didx, ones_v,
              s0, s1, s2, s3):
  cid = lax.axis_index("c")
  sid = lax.axis_index("s")
  wid = cid * NS + sid
  sems = (s0, s1, s2, s3)

  pltpu.sync_copy(zs_hbm.at[pl.ds(sid * 640, 640)],
                  deg_sp.at[pl.ds(sid * 640, 640)])
  pltpu.sync_copy(dst_hbm.at[wid], didx)
  for i in range(8):
    ones_v[pl.ds(i * 16, 16)] = jnp.full((16,), 1.0, jnp.float32)
  plsc.subcore_barrier()

  def body(j, carry):
    descs = []
    for p in range(DSLOT):
      descs.append(pltpu.async_copy(
          ones_v, deg_sp.at[didx.at[j * DSLOT + p]], sems[p], add=True))
    for d in descs:
      d.wait()
    return carry

  lax.fori_loop(0, NCH // DSLOT, body, 0)

  plsc.subcore_barrier()
  pltpu.sync_copy(deg_sp.at[pl.ds(sid * 640, 640)],
                  deg_hbm.at[cid, pl.ds(sid * 640, 640)])


_deg = pl.kernel(
    _deg_body,
    out_type=jax.ShapeDtypeStruct((NC, NPAD), jnp.float32),
    mesh=_MESH,
    compiler_params=_SC_PARAMS,
    scratch_types=[
        pltpu.VMEM_SHARED((NPAD,), jnp.float32),
        pltpu.VMEM((NCH, CHUNK), jnp.int32),
        pltpu.VMEM((CHUNK,), jnp.float32),
        pltpu.SemaphoreType.DMA,
        pltpu.SemaphoreType.DMA,
        pltpu.SemaphoreType.DMA,
        pltpu.SemaphoreType.DMA,
    ],
)


def _proj_body(x_ref, w_ref, degp_ref, g_ref, dis_ref):
  deg = degp_ref[0] + degp_ref[1] + 1.0
  dis = lax.rsqrt(deg)
  dis_ref[...] = dis
  h = jnp.dot(x_ref[...], w_ref[...], preferred_element_type=jnp.float32)
  g_ref[...] = dis * h


def _proj(x, W1, degp):
  bm = 2000
  return pl.pallas_call(
      _proj_body,
      grid=(N_NODES // bm,),
      in_specs=[
          pl.BlockSpec((bm, D), lambda i: (i, 0)),
          pl.BlockSpec((D, D), lambda i: (0, 0)),
          pl.BlockSpec((NC, bm, 1), lambda i: (0, i, 0)),
      ],
      out_specs=[
          pl.BlockSpec((bm, D), lambda i: (i, 0)),
          pl.BlockSpec((bm, 1), lambda i: (i, 0)),
      ],
      out_shape=[
          jax.ShapeDtypeStruct((N_NODES, D), jnp.float32),
          jax.ShapeDtypeStruct((N_NODES, 1), jnp.float32),
      ],
  )(x, W1, degp)


def _edge_body(src_hbm, dst_hbm, dis_hbm, g_hbm, zrow_hbm, zs_hbm,
               sp_hbm, accp_hbm,
               acc_sp, s_sp, dis_v, srcb, dstb, val_v, row_v,
               g0, g1, t0, t1, i0, i1, i2, i3, asem):
  cid = lax.axis_index("c")
  sid = lax.axis_index("s")
  wid = cid * NS + sid
  gsem = (g0, g1)
  ssem = (t0, t1)
  isem = (i0, i1, i2, i3)

  def issue_idx(c, q):
    pltpu.async_copy(src_hbm.at[wid, c], srcb.at[q], isem[q])
    pltpu.async_copy(dst_hbm.at[wid, c], dstb.at[q], isem[q])

  def wait_idx(c, q):
    pltpu.make_async_copy(src_hbm.at[wid, c], srcb.at[q], isem[q]).wait()
    pltpu.make_async_copy(dst_hbm.at[wid, c], dstb.at[q], isem[q]).wait()

  pltpu.sync_copy(zrow_hbm.at[pl.ds(sid * 640, 640)],
                  acc_sp.at[pl.ds(sid * 640, 640)])
  pltpu.sync_copy(zs_hbm.at[pl.ds(sid * 640, 640)],
                  s_sp.at[pl.ds(sid * 640, 640)])
  pltpu.sync_copy(dis_hbm, dis_v)
  plsc.subcore_barrier()

  for q in range(ISLOT):
    issue_idx(q, q)
  for p in range(NSLOT):
    wait_idx(p, p)
    pltpu.async_copy(g_hbm.at[srcb.at[p]], row_v.at[p], gsem[p])

  def body(j, carry):
    for u in range(ISLOT):
      c = j * ISLOT + u
      p = u % NSLOT
      q = u
      pltpu.make_async_copy(
          g_hbm.at[srcb.at[q]], row_v.at[p], gsem[p]).wait()
      sdesc = pltpu.async_copy(
          row_v.at[p], acc_sp.at[dstb.at[q]], ssem[p], add=True)
      dstb_row = dstb.at[q]
      for k in range(8):
        dv = dstb_row[pl.ds(k * 16, 16)]
        val_v[pl.ds(k * 16, 16)] = plsc.load_gather(dis_v, [dv])
      asdesc = pltpu.async_copy(val_v, s_sp.at[srcb.at[q]], asem,
                                add=True)
      sdesc.wait()

      @pl.when(c + NSLOT < NCH)
      def _():
        qn = (u + NSLOT) % ISLOT
        wait_idx(c + NSLOT, qn)
        pltpu.async_copy(g_hbm.at[srcb.at[qn]], row_v.at[p], gsem[p])

      asdesc.wait()

      @pl.when(c + ISLOT < NCH)
      def _():
        issue_idx(c + ISLOT, q)
    return carry

  lax.fori_loop(0, NCH // ISLOT, body, 0)

  plsc.subcore_barrier()
  pltpu.sync_copy(acc_sp.at[pl.ds(sid * 640, 640)],
                  accp_hbm.at[cid, pl.ds(sid * 640, 640)])
  pltpu.sync_copy(s_sp.at[pl.ds(sid * 640, 640)],
                  sp_hbm.at[cid, pl.ds(sid * 640, 640)])


_edge_pass = pl.kernel(
    _edge_body,
    out_type=(
        jax.ShapeDtypeStruct((NC, NPAD), jnp.float32),
        jax.ShapeDtypeStruct((NC, NPAD, D), jnp.float32),
    ),
    mesh=_MESH,
    compiler_params=_SC_PARAMS,
    scratch_types=[
        pltpu.VMEM_SHARED((NPAD, D), jnp.float32),
        pltpu.VMEM_SHARED((NPAD,), jnp.float32),
        pltpu.VMEM((NDIS,), jnp.float32),
        pltpu.VMEM((ISLOT, CHUNK), jnp.int32),
        pltpu.VMEM((ISLOT, CHUNK), jnp.int32),
        pltpu.VMEM((CHUNK,), jnp.float32),
        pltpu.VMEM((NSLOT, CHUNK, D), jnp.float32),
        pltpu.SemaphoreType.DMA,
        pltpu.SemaphoreType.DMA,
        pltpu.SemaphoreType.DMA,
        pltpu.SemaphoreType.DMA,
        pltpu.SemaphoreType.DMA,
        pltpu.SemaphoreType.DMA,
        pltpu.SemaphoreType.DMA,
        pltpu.SemaphoreType.DMA,
        pltpu.SemaphoreType.DMA,
    ],
)


def _head_body(dis_ref, g_ref, accp_ref, sp_ref, b1_ref, w2_ref, b2_ref,
               wfc_ref, bfc_ref, out_ref, vacc):
  i = pl.program_id(0)
  dis = dis_ref[...]
  acc = accp_ref[0] + accp_ref[1]
  z1 = dis * (acc + g_ref[...]) + b1_ref[...]
  r1 = jnp.maximum(z1, 0.0)
  s = sp_ref[0] + sp_ref[1]
  w = dis * (dis + s)
  contrib = jnp.sum(w * r1, axis=0, keepdims=True)

  @pl.when(i == 0)
  def _():
    vacc[...] = jnp.zeros_like(vacc)

  vacc[...] += contrib

  @pl.when(i == pl.num_programs(0) - 1)
  def _():
    pooled = jnp.dot(vacc[...] * (1.0 / N_NODES), w2_ref[...],
                     preferred_element_type=jnp.float32) + b2_ref[...]
    out_ref[...] = jnp.dot(pooled, wfc_ref[...],
                           preferred_element_type=jnp.float32) + bfc_ref[...]


def _head(dis_col, g, accp, sp, b1, W2, b2, Wfc, bfc):
  bm = 2000
  return pl.pallas_call(
      _head_body,
      grid=(N_NODES // bm,),
      in_specs=[
          pl.BlockSpec((bm, 1), lambda i: (i, 0)),
          pl.BlockSpec((bm, D), lambda i: (i, 0)),
          pl.BlockSpec((NC, bm, D), lambda i: (0, i, 0)),
          pl.BlockSpec((NC, bm, 1), lambda i: (0, i, 0)),
          pl.BlockSpec((1, D), lambda i: (0, 0)),
          pl.BlockSpec((D, D), lambda i: (0, 0)),
          pl.BlockSpec((1, D), lambda i: (0, 0)),
          pl.BlockSpec((D, D), lambda i: (0, 0)),
          pl.BlockSpec((1, D), lambda i: (0, 0)),
      ],
      out_specs=pl.BlockSpec((1, D), lambda i: (0, 0)),
      out_shape=jax.ShapeDtypeStruct((1, D), jnp.float32),
      scratch_shapes=[pltpu.VMEM((1, D), jnp.float32)],
  )(dis_col, g, accp, sp, b1, W2, b2, Wfc, bfc)


def kernel(x, edge_index, W1, b1, W2, b2, Wfc, bfc):
  src = edge_index[0].astype(jnp.int32)
  dst = edge_index[1].astype(jnp.int32)
  npad_e = E_PAD - N_EDGES
  src_p = jnp.concatenate(
      [src, jnp.zeros((npad_e,), jnp.int32)]).reshape(NW, NCH, CHUNK)
  dst_p = jnp.concatenate(
      [dst, jnp.full((npad_e,), DEAD, jnp.int32)]).reshape(NW, NCH, CHUNK)
  zrow = jnp.zeros((NPAD, D), jnp.float32)
  zs = jnp.zeros((NPAD,), jnp.float32)

  degp = _deg(dst_p, zs)
  g, dis_col = _proj(x, W1, degp.reshape(NC, NPAD, 1))
  dis_pad = jnp.concatenate(
      [dis_col.reshape(N_NODES), jnp.zeros((NDIS - N_NODES,), jnp.float32)])
  sp, accp = _edge_pass(src_p, dst_p, dis_pad, g, zrow, zs)
  sp_col = sp.reshape(NC, NPAD, 1)
  b1r = b1.reshape(1, D)
  b2r = b2.reshape(1, D)
  bfcr = bfc.reshape(1, D)
  return _head(dis_col, g, accp, sp_col, b1r, W2, b2r, Wfc, bfcr)


# speedup vs baseline: 1.0006x; 1.0002x over previous
import jax
import jax.numpy as jnp
from jax import lax
from jax.experimental import pallas as pl
from jax.experimental.pallas import tpu as pltpu
from jax.experimental.pallas import tpu_sc as plsc

N_NODES = 10000
N_EDGES = 320000
D = 128
NPAD = 10240
NDIS = 10016
DEAD = 10008
NC = 2
NS = 16
NW = NC * NS
CHUNK = 128
NCH = 80
EPT = NCH * CHUNK
E_PAD = NW * EPT
NSLOT = 2
ISLOT = 4
DSLOT = 4

_MESH = plsc.VectorSubcoreMesh(core_axis_name="c", subcore_axis_name="s")
_SC_PARAMS = pltpu.CompilerParams(needs_layout_passes=False)


def _deg_body(dst_hbm, zs_hbm, deg_hbm, deg_sp, didx, ones_v,
              s0, s1, s2, s3):
  cid = lax.axis_index("c")
  sid = lax.axis_index("s")
  wid = cid * NS + sid
  sems = (s0, s1, s2, s3)

  pltpu.sync_copy(zs_hbm.at[pl.ds(sid * 640, 640)],
                  deg_sp.at[pl.ds(sid * 640, 640)])
  pltpu.sync_copy(dst_hbm.at[wid], didx)
  for i in range(8):
    ones_v[pl.ds(i * 16, 16)] = jnp.full((16,), 1.0, jnp.float32)
  plsc.subcore_barrier()

  def body(j, carry):
    descs = []
    for p in range(DSLOT):
      descs.append(pltpu.async_copy(
          ones_v, deg_sp.at[didx.at[j * DSLOT + p]], sems[p], add=True))
    for d in descs:
      d.wait()
    return carry

  lax.fori_loop(0, NCH // DSLOT, body, 0)

  plsc.subcore_barrier()
  pltpu.sync_copy(deg_sp.at[pl.ds(sid * 640, 640)],
                  deg_hbm.at[cid, pl.ds(sid * 640, 640)])


_deg = pl.kernel(
    _deg_body,
    out_type=jax.ShapeDtypeStruct((NC, NPAD), jnp.float32),
    mesh=_MESH,
    compiler_params=_SC_PARAMS,
    scratch_types=[
        pltpu.VMEM_SHARED((NPAD,), jnp.float32),
        pltpu.VMEM((NCH, CHUNK), jnp.int32),
        pltpu.VMEM((CHUNK,), jnp.float32),
        pltpu.SemaphoreType.DMA,
        pltpu.SemaphoreType.DMA,
        pltpu.SemaphoreType.DMA,
        pltpu.SemaphoreType.DMA,
    ],
)


def _proj_body(x_ref, w_ref, degp_ref, g_ref, dis_ref):
  deg = degp_ref[0] + degp_ref[1] + 1.0
  dis = lax.rsqrt(deg)
  dis_ref[...] = dis
  h = jnp.dot(x_ref[...], w_ref[...], preferred_element_type=jnp.float32)
  g_ref[...] = dis * h


def _proj(x, W1, degp):
  bm = 2000
  return pl.pallas_call(
      _proj_body,
      grid=(N_NODES // bm,),
      in_specs=[
          pl.BlockSpec((bm, D), lambda i: (i, 0)),
          pl.BlockSpec((D, D), lambda i: (0, 0)),
          pl.BlockSpec((NC, bm, 1), lambda i: (0, i, 0)),
      ],
      out_specs=[
          pl.BlockSpec((bm, D), lambda i: (i, 0)),
          pl.BlockSpec((bm, 1), lambda i: (i, 0)),
      ],
      out_shape=[
          jax.ShapeDtypeStruct((N_NODES, D), jnp.float32),
          jax.ShapeDtypeStruct((N_NODES, 1), jnp.float32),
      ],
  )(x, W1, degp)


def _edge_body(src_hbm, dst_hbm, dis_hbm, g_hbm, zrow_hbm, zs_hbm,
               sp_hbm, accp_hbm,
               acc_sp, s_sp, dis_v, srcb, dstb, val_v, row_v,
               g0, g1, t0, t1, i0, i1, i2, i3, asem):
  cid = lax.axis_index("c")
  sid = lax.axis_index("s")
  wid = cid * NS + sid
  gsem = (g0, g1)
  ssem = (t0, t1)
  isem = (i0, i1, i2, i3)

  def issue_idx(c, q):
    pltpu.async_copy(src_hbm.at[wid, c], srcb.at[q], isem[q])
    pltpu.async_copy(dst_hbm.at[wid, c], dstb.at[q], isem[q])

  def wait_idx(c, q):
    pltpu.make_async_copy(src_hbm.at[wid, c], srcb.at[q], isem[q]).wait()
    pltpu.make_async_copy(dst_hbm.at[wid, c], dstb.at[q], isem[q]).wait()

  pltpu.sync_copy(zrow_hbm.at[pl.ds(sid * 640, 640)],
                  acc_sp.at[pl.ds(sid * 640, 640)])
  pltpu.sync_copy(zs_hbm.at[pl.ds(sid * 640, 640)],
                  s_sp.at[pl.ds(sid * 640, 640)])
  pltpu.sync_copy(dis_hbm, dis_v)
  plsc.subcore_barrier()

  def issue_gather(q, p):
    pltpu.async_copy(g_hbm.at[srcb.at[q, pl.ds(0, 64)]],
                     row_v.at[p, pl.ds(0, 64)], gsem[p])
    pltpu.async_copy(g_hbm.at[srcb.at[q, pl.ds(64, 64)]],
                     row_v.at[p, pl.ds(64, 64)], gsem[p])

  def wait_gather(q, p):
    pltpu.make_async_copy(g_hbm.at[srcb.at[q, pl.ds(0, 64)]],
                          row_v.at[p, pl.ds(0, 64)], gsem[p]).wait()
    pltpu.make_async_copy(g_hbm.at[srcb.at[q, pl.ds(64, 64)]],
                          row_v.at[p, pl.ds(64, 64)], gsem[p]).wait()

  for q in range(ISLOT):
    issue_idx(q, q)
  for p in range(NSLOT):
    wait_idx(p, p)
    issue_gather(p, p)

  def body(j, carry):
    for u in range(ISLOT):
      c = j * ISLOT + u
      p = u % NSLOT
      q = u
      wait_gather(q, p)
      sdesc = pltpu.async_copy(
          row_v.at[p, pl.ds(0, 64)],
          acc_sp.at[dstb.at[q, pl.ds(0, 64)]], ssem[p], add=True)
      sdesc2 = pltpu.async_copy(
          row_v.at[p, pl.ds(64, 64)],
          acc_sp.at[dstb.at[q, pl.ds(64, 64)]], ssem[p], add=True)
      dstb_row = dstb.at[q]
      for k in range(8):
        dv = dstb_row[pl.ds(k * 16, 16)]
        val_v[pl.ds(k * 16, 16)] = plsc.load_gather(dis_v, [dv])
      asdesc = pltpu.async_copy(val_v, s_sp.at[srcb.at[q]], asem,
                                add=True)
      sdesc.wait()
      sdesc2.wait()

      @pl.when(c + NSLOT < NCH)
      def _():
        qn = (u + NSLOT) % ISLOT
        wait_idx(c + NSLOT, qn)
        issue_gather(qn, p)

      asdesc.wait()

      @pl.when(c + ISLOT < NCH)
      def _():
        issue_idx(c + ISLOT, q)
    return carry

  lax.fori_loop(0, NCH // ISLOT, body, 0)

  plsc.subcore_barrier()
  pltpu.sync_copy(acc_sp.at[pl.ds(sid * 640, 640)],
                  accp_hbm.at[cid, pl.ds(sid * 640, 640)])
  pltpu.sync_copy(s_sp.at[pl.ds(sid * 640, 640)],
                  sp_hbm.at[cid, pl.ds(sid * 640, 640)])


_edge_pass = pl.kernel(
    _edge_body,
    out_type=(
        jax.ShapeDtypeStruct((NC, NPAD), jnp.float32),
        jax.ShapeDtypeStruct((NC, NPAD, D), jnp.float32),
    ),
    mesh=_MESH,
    compiler_params=_SC_PARAMS,
    scratch_types=[
        pltpu.VMEM_SHARED((NPAD, D), jnp.float32),
        pltpu.VMEM_SHARED((NPAD,), jnp.float32),
        pltpu.VMEM((NDIS,), jnp.float32),
        pltpu.VMEM((ISLOT, CHUNK), jnp.int32),
        pltpu.VMEM((ISLOT, CHUNK), jnp.int32),
        pltpu.VMEM((CHUNK,), jnp.float32),
        pltpu.VMEM((NSLOT, CHUNK, D), jnp.float32),
        pltpu.SemaphoreType.DMA,
        pltpu.SemaphoreType.DMA,
        pltpu.SemaphoreType.DMA,
        pltpu.SemaphoreType.DMA,
        pltpu.SemaphoreType.DMA,
        pltpu.SemaphoreType.DMA,
        pltpu.SemaphoreType.DMA,
        pltpu.SemaphoreType.DMA,
        pltpu.SemaphoreType.DMA,
    ],
)


def _head_body(dis_ref, g_ref, accp_ref, sp_ref, b1_ref, w2_ref, b2_ref,
               wfc_ref, bfc_ref, out_ref, vacc):
  i = pl.program_id(0)
  dis = dis_ref[...]
  acc = accp_ref[0] + accp_ref[1]
  z1 = dis * (acc + g_ref[...]) + b1_ref[...]
  r1 = jnp.maximum(z1, 0.0)
  s = sp_ref[0] + sp_ref[1]
  w = dis * (dis + s)
  contrib = jnp.sum(w * r1, axis=0, keepdims=True)

  @pl.when(i == 0)
  def _():
    vacc[...] = jnp.zeros_like(vacc)

  vacc[...] += contrib

  @pl.when(i == pl.num_programs(0) - 1)
  def _():
    pooled = jnp.dot(vacc[...] * (1.0 / N_NODES), w2_ref[...],
                     preferred_element_type=jnp.float32) + b2_ref[...]
    out_ref[...] = jnp.dot(pooled, wfc_ref[...],
                           preferred_element_type=jnp.float32) + bfc_ref[...]


def _head(dis_col, g, accp, sp, b1, W2, b2, Wfc, bfc):
  bm = 2000
  return pl.pallas_call(
      _head_body,
      grid=(N_NODES // bm,),
      in_specs=[
          pl.BlockSpec((bm, 1), lambda i: (i, 0)),
          pl.BlockSpec((bm, D), lambda i: (i, 0)),
          pl.BlockSpec((NC, bm, D), lambda i: (0, i, 0)),
          pl.BlockSpec((NC, bm, 1), lambda i: (0, i, 0)),
          pl.BlockSpec((1, D), lambda i: (0, 0)),
          pl.BlockSpec((D, D), lambda i: (0, 0)),
          pl.BlockSpec((1, D), lambda i: (0, 0)),
          pl.BlockSpec((D, D), lambda i: (0, 0)),
          pl.BlockSpec((1, D), lambda i: (0, 0)),
      ],
      out_specs=pl.BlockSpec((1, D), lambda i: (0, 0)),
      out_shape=jax.ShapeDtypeStruct((1, D), jnp.float32),
      scratch_shapes=[pltpu.VMEM((1, D), jnp.float32)],
  )(dis_col, g, accp, sp, b1, W2, b2, Wfc, bfc)


def kernel(x, edge_index, W1, b1, W2, b2, Wfc, bfc):
  src = edge_index[0].astype(jnp.int32)
  dst = edge_index[1].astype(jnp.int32)
  npad_e = E_PAD - N_EDGES
  src_p = jnp.concatenate(
      [src, jnp.zeros((npad_e,), jnp.int32)]).reshape(NW, NCH, CHUNK)
  dst_p = jnp.concatenate(
      [dst, jnp.full((npad_e,), DEAD, jnp.int32)]).reshape(NW, NCH, CHUNK)
  zrow = jnp.zeros((NPAD, D), jnp.float32)
  zs = jnp.zeros((NPAD,), jnp.float32)

  degp = _deg(dst_p, zs)
  g, dis_col = _proj(x, W1, degp.reshape(NC, NPAD, 1))
  dis_pad = jnp.concatenate(
      [dis_col.reshape(N_NODES), jnp.zeros((NDIS - N_NODES,), jnp.float32)])
  sp, accp = _edge_pass(src_p, dst_p, dis_pad, g, zrow, zs)
  sp_col = sp.reshape(NC, NPAD, 1)
  b1r = b1.reshape(1, D)
  b2r = b2.reshape(1, D)
  bfcr = bfc.reshape(1, D)
  return _head(dis_col, g, accp, sp_col, b1r, W2, b2r, Wfc, bfcr)
